# baseline (device time: 222900 ns/iter reference)
import jax
import jax.numpy as jnp
from jax import lax
from jax.experimental import pallas as pl
from jax.experimental.pallas import tpu as pltpu

R = 8


def kernel(A, B):
    M, K = A.shape
    _, N = B.shape
    m_blk = M // R

    A = A.astype(jnp.bfloat16)
    B = B.astype(jnp.bfloat16)

    def body(a_ref, b_ref, out_ref,
             sendbuf, recvbuf, send_sems, recv_sems, credit_sem):
        me = lax.axis_index("i")
        right = lax.rem(me + 1, R)
        left = lax.rem(me + R - 1, R)

        barrier = pltpu.get_barrier_semaphore()
        for nbr in (left, right):
            pl.semaphore_signal(barrier, inc=1, device_id=(nbr,),
                                device_id_type=pl.DeviceIdType.MESH)
        pl.semaphore_wait(barrier, 2)

        def partial(b):
            return jnp.dot(a_ref[pl.ds(b * m_blk, m_blk), :], b_ref[...],
                           preferred_element_type=jnp.float32)

        b0 = lax.rem(me + R - 1, R)
        sendbuf[0] = partial(b0).astype(jnp.bfloat16)

        for h in range(R - 1):
            if h >= 2:
                pl.semaphore_wait(credit_sem, 1)
            rdma = pltpu.make_async_remote_copy(
                src_ref=sendbuf.at[h % 2],
                dst_ref=recvbuf.at[h % 2],
                send_sem=send_sems.at[h % 2],
                recv_sem=recv_sems.at[h % 2],
                device_id=(right,),
                device_id_type=pl.DeviceIdType.MESH,
            )
            rdma.start()
            bh = lax.rem(me + 2 * R - 2 - h, R)
            p = partial(bh)
            rdma.wait()
            if h < R - 2:
                sendbuf[(h + 1) % 2] = (
                    recvbuf[h % 2].astype(jnp.float32) + p
                ).astype(jnp.bfloat16)
            else:
                out_ref[...] = recvbuf[h % 2].astype(jnp.float32) + p
            if h <= R - 4:
                pl.semaphore_signal(credit_sem, inc=1, device_id=(left,),
                                    device_id_type=pl.DeviceIdType.MESH)

    out_shape = jax.ShapeDtypeStruct((m_blk, N), jnp.float32)
    return pl.pallas_call(
        body,
        out_shape=out_shape,
        in_specs=[pl.BlockSpec(memory_space=pltpu.VMEM),
                  pl.BlockSpec(memory_space=pltpu.VMEM)],
        out_specs=pl.BlockSpec(memory_space=pltpu.VMEM),
        scratch_shapes=[
            pltpu.VMEM((2, m_blk, N), jnp.bfloat16),
            pltpu.VMEM((2, m_blk, N), jnp.bfloat16),
            pltpu.SemaphoreType.DMA((2,)),
            pltpu.SemaphoreType.DMA((2,)),
            pltpu.SemaphoreType.REGULAR,
        ],
        compiler_params=pltpu.CompilerParams(collective_id=0),
    )(A, B)


# device time: 59309 ns/iter; 3.7583x vs baseline; 3.7583x over previous
import jax
import jax.numpy as jnp
from jax import lax
from jax.experimental import pallas as pl
from jax.experimental.pallas import tpu as pltpu

R = 8
ORDERS = ((4, 3, 1), (3, 1, 4), (1, 4, 3))


def kernel(A, B):
    M, K = A.shape
    _, N = B.shape
    m_blk = M // R
    n_q = N // 3

    A = A.astype(jnp.bfloat16)
    B = B.astype(jnp.bfloat16)

    def body(a_ref, b_ref, out_ref,
             sbuf1, rbuf1, sbuf2, rbuf2, sbuf3, rbuf3,
             ssem1, rsem1, ssem2, rsem2, ssem3, rsem3):
        me = lax.axis_index("i")

        barrier = pltpu.get_barrier_semaphore()
        for mask in (1, 3, 4):
            pl.semaphore_signal(barrier, inc=1, device_id=(me ^ mask,),
                                device_id_type=pl.DeviceIdType.MESH)
        pl.semaphore_wait(barrier, 3)

        def partial(b, q):
            return jnp.dot(
                a_ref[pl.ds(b * m_blk, m_blk), :],
                b_ref[:, q * n_q:(q + 1) * n_q],
                preferred_element_type=jnp.float32,
            )

        def start(q, src, dst, ssem, rsem, mask):
            rdma = pltpu.make_async_remote_copy(
                src_ref=src.at[q], dst_ref=dst.at[q],
                send_sem=ssem.at[q], recv_sem=rsem.at[q],
                device_id=(me ^ mask,),
                device_id_type=pl.DeviceIdType.MESH,
            )
            rdma.start()
            return rdma

        for q, (X1, X2, X3) in enumerate(ORDERS):
            for j, d in enumerate((0, X2, X3, X2 ^ X3)):
                sbuf1[q, j] = partial(me ^ X1 ^ d, q).astype(jnp.bfloat16)
        rd1 = [start(q, sbuf1, rbuf1, ssem1, rsem1, o[0])
               for q, o in enumerate(ORDERS)]

        for q, (X1, X2, X3) in enumerate(ORDERS):
            sbuf2[q, 0] = partial(me ^ X2, q).astype(jnp.bfloat16)
            sbuf2[q, 1] = partial(me ^ X2 ^ X3, q).astype(jnp.bfloat16)
        for r in rd1:
            r.wait()

        for q in range(3):
            sbuf2[q, 0] = sbuf2[q, 0] + rbuf1[q, 1]
            sbuf2[q, 1] = sbuf2[q, 1] + rbuf1[q, 3]
        rd2 = [start(q, sbuf2, rbuf2, ssem2, rsem2, o[1])
               for q, o in enumerate(ORDERS)]

        for q, (X1, X2, X3) in enumerate(ORDERS):
            sbuf3[q, 0] = partial(me ^ X3, q).astype(jnp.bfloat16)
        for r in rd2:
            r.wait()

        for q in range(3):
            sbuf3[q, 0] = sbuf3[q, 0] + rbuf1[q, 2] + rbuf2[q, 1]
        rd3 = [start(q, sbuf3, rbuf3, ssem3, rsem3, o[2])
               for q, o in enumerate(ORDERS)]

        own = [partial(me, q)
               + rbuf1[q, 0].astype(jnp.float32)
               + rbuf2[q, 0].astype(jnp.float32)
               for q in range(3)]
        for r in rd3:
            r.wait()

        for q in range(3):
            out_ref[:, q * n_q:(q + 1) * n_q] = (
                own[q] + rbuf3[q, 0].astype(jnp.float32)
            )

    out_shape = jax.ShapeDtypeStruct((m_blk, N), jnp.float32)
    bf = jnp.bfloat16
    return pl.pallas_call(
        body,
        out_shape=out_shape,
        in_specs=[pl.BlockSpec(memory_space=pltpu.VMEM),
                  pl.BlockSpec(memory_space=pltpu.VMEM)],
        out_specs=pl.BlockSpec(memory_space=pltpu.VMEM),
        scratch_shapes=[
            pltpu.VMEM((3, 4, m_blk, n_q), bf),
            pltpu.VMEM((3, 4, m_blk, n_q), bf),
            pltpu.VMEM((3, 2, m_blk, n_q), bf),
            pltpu.VMEM((3, 2, m_blk, n_q), bf),
            pltpu.VMEM((3, 1, m_blk, n_q), bf),
            pltpu.VMEM((3, 1, m_blk, n_q), bf),
            pltpu.SemaphoreType.DMA((3,)),
            pltpu.SemaphoreType.DMA((3,)),
            pltpu.SemaphoreType.DMA((3,)),
            pltpu.SemaphoreType.DMA((3,)),
            pltpu.SemaphoreType.DMA((3,)),
            pltpu.SemaphoreType.DMA((3,)),
        ],
        compiler_params=pltpu.CompilerParams(
            collective_id=0,
            vmem_limit_bytes=100 * 1024 * 1024,
        ),
    )(A, B)
